# baseline (speedup 1.0000x reference)
import functools

import jax
import jax.numpy as jnp
from jax import lax
from jax.experimental import pallas as pl
from jax.experimental.pallas import tpu as pltpu
from jax.experimental.pallas import tpu_sc as plsc

N, E, D, DE = 10000, 320000, 128, 16
H2 = D // 2
NC, NS = 2, 16
NW = NC * NS
EPW = E // NW
GB = 80
GK = EPW // GB
BN = 2000
BE = 2560
NEB = E // BE
RPT = 1000
NST = N // RPT
S = 5
ES = E // S
EPWS = EPW // S
GKS = GK // S
NEBS = NEB // S
ETS = ES // NS
GKT = ETS // GB

_f32 = jnp.float32
_i32 = jnp.int32


def _tables_body(h_ref, wcat_ref, bcat_ref, b_ref, pq_ref):
    x = jnp.dot(h_ref[...], wcat_ref[...], preferred_element_type=_f32)
    x = x + bcat_ref[...]
    b_ref[...] = x[:, :D]
    pq_ref[...] = x[:, D:]


def _tables(h, wcat, bcat):
    return pl.pallas_call(
        _tables_body,
        grid=(N // BN,),
        in_specs=[
            pl.BlockSpec((BN, D), lambda i: (i, 0)),
            pl.BlockSpec((D, 2 * D), lambda i: (0, 0)),
            pl.BlockSpec((1, 2 * D), lambda i: (0, 0)),
        ],
        out_specs=[
            pl.BlockSpec((BN, D), lambda i: (i, 0)),
            pl.BlockSpec((BN, D), lambda i: (i, 0)),
        ],
        out_shape=[
            jax.ShapeDtypeStruct((N, D), _f32),
            jax.ShapeDtypeStruct((N, D), _f32),
        ],
    )(h, wcat, bcat)


def _gather_sc(b_tab, pq_tab, src3, dst3):
    mesh = plsc.VectorSubcoreMesh(core_axis_name="c", subcore_axis_name="s")

    @functools.partial(
        pl.kernel,
        mesh=mesh,
        out_type=[
            jax.ShapeDtypeStruct((ES, D), _f32),
            jax.ShapeDtypeStruct((ES, D), _f32),
            jax.ShapeDtypeStruct((ES, D), _f32),
        ],
        scratch_types=[
            pltpu.VMEM((GKS, GB), _i32),
            pltpu.VMEM((GKS, GB), _i32),
            pltpu.VMEM((2, GB, D), _f32),
            pltpu.VMEM((2, GB, D), _f32),
            pltpu.VMEM((2, GB, D), _f32),
            pltpu.SemaphoreType.DMA,
            pltpu.SemaphoreType.DMA,
        ],
    )
    def gather_k(b_hbm, pq_hbm, src_hbm, dst_hbm, bs_out, pqs_out, pqd_out,
                 src_v, dst_v, bs_v, pqs_v, pqd_v, sem0, sem1):
        wid = lax.axis_index("s") * NC + lax.axis_index("c")
        pltpu.sync_copy(src_hbm.at[wid], src_v)
        pltpu.sync_copy(dst_hbm.at[wid], dst_v)
        sems = (sem0, sem1)

        def fire(j, par):
            sem = sems[par]
            pltpu.async_copy(b_hbm.at[src_v.at[j]], bs_v.at[par], sem)
            pltpu.async_copy(pq_hbm.at[src_v.at[j]], pqs_v.at[par], sem)
            pltpu.async_copy(pq_hbm.at[dst_v.at[j]], pqd_v.at[par], sem)

        def drain(j, par):
            dummy = b_hbm.at[pl.ds(0, GB)]
            for dst in (bs_v.at[par], pqs_v.at[par], pqd_v.at[par]):
                pltpu.make_async_copy(dummy, dst, sems[par]).wait()
            base = wid * EPWS + j * GB
            pltpu.sync_copy(bs_v.at[par], bs_out.at[pl.ds(base, GB)])
            pltpu.sync_copy(pqs_v.at[par], pqs_out.at[pl.ds(base, GB)])
            pltpu.sync_copy(pqd_v.at[par], pqd_out.at[pl.ds(base, GB)])

        fire(0, 0)

        def body(k, carry):
            j0 = 2 * k
            fire(j0 + 1, 1)
            drain(j0, 0)
            fire(j0 + 2, 0)
            drain(j0 + 1, 1)
            return carry

        lax.fori_loop(0, (GKS - 1) // 2, body, 0)
        drain(GKS - 1, 0)

    return gather_k(b_tab, pq_tab, src3, dst3)


def _edge_body(bs_ref, pqs_ref, pqd_ref, ea_ref, mW1b_ref, aW1e_ref, ab1_ref,
               aW2_ref, ab2_ref, ext_ref, t8_ref, bmax_ref):
    ea = ea_ref[...]
    cem = jnp.dot(ea, mW1b_ref[...], preferred_element_type=_f32)
    cea = jnp.dot(ea, aW1e_ref[...], preferred_element_type=_f32)
    mh = jnp.maximum(bs_ref[...] + cem, 0.0)
    pre = pqs_ref[:, :H2] + pqd_ref[:, H2:] + cea + ab1_ref[...]
    ah = jnp.maximum(pre, 0.2 * pre)
    sc = jnp.sum(ah * aW2_ref[...], axis=1, keepdims=True) + ab2_ref[...]
    t = jnp.exp(sc)
    ext_ref[...] = mh * t
    t8_ref[...] = jnp.broadcast_to(t, (BE, 8))
    bmax_ref[...] = jnp.full((1, 1, 128), jnp.max(sc), dtype=_f32)


def _edge_stage(bs, pqs, pqd, ea, mW1b, aW1e, ab1r, aW2r, ab2r):
    return pl.pallas_call(
        _edge_body,
        grid=(NEBS,),
        in_specs=[
            pl.BlockSpec((BE, D), lambda i: (i, 0)),
            pl.BlockSpec((BE, D), lambda i: (i, 0)),
            pl.BlockSpec((BE, D), lambda i: (i, 0)),
            pl.BlockSpec((BE, DE), lambda i: (i, 0)),
            pl.BlockSpec((DE, D), lambda i: (0, 0)),
            pl.BlockSpec((DE, H2), lambda i: (0, 0)),
            pl.BlockSpec((1, H2), lambda i: (0, 0)),
            pl.BlockSpec((1, H2), lambda i: (0, 0)),
            pl.BlockSpec((1, 1), lambda i: (0, 0)),
        ],
        out_specs=[
            pl.BlockSpec((BE, D), lambda i: (i, 0)),
            pl.BlockSpec((BE, 8), lambda i: (i, 0)),
            pl.BlockSpec((1, 1, 128), lambda i: (i, 0, 0)),
        ],
        out_shape=[
            jax.ShapeDtypeStruct((ES, D), _f32),
            jax.ShapeDtypeStruct((ES, 8), _f32),
            jax.ShapeDtypeStruct((NEBS, 1, 128), _f32),
        ],
    )(bs, pqs, pqd, ea, mW1b, aW1e, ab1r, aW2r, ab2r)


EPT = E // NS


def _scatter_sc(exts, ts, dst_flat, zeros):
    mesh = plsc.VectorSubcoreMesh(core_axis_name="c", subcore_axis_name="s")

    @functools.partial(
        pl.kernel,
        mesh=mesh,
        out_type=jax.ShapeDtypeStruct((2 * N, D), _f32),
        scratch_types=[
            pltpu.VMEM((2, GB), _i32),
            pltpu.VMEM((2, GB, D), _f32),
            pltpu.VMEM((2, GB), _f32),
            pltpu.VMEM_SHARED((N, D), _f32),
            pltpu.SemaphoreType.DMA,
            pltpu.SemaphoreType.DMA,
            pltpu.SemaphoreType.DMA,
            pltpu.SemaphoreType.DMA,
        ],
    )
    def scatter_k(e0, e1, e2, e3, e4, t0, t1, t2, t3, t4, dst_hbm, zeros_hbm,
                  out_hbm, didx_v, rows_v, t_v, acc_sh, sem0, sem1, xem0, xem1):
        ehs = (e0, e1, e2, e3, e4)
        ths = (t0, t1, t2, t3, t4)
        cid = lax.axis_index("c")
        sid = lax.axis_index("s")
        sems = (sem0, sem1)
        xems = (xem0, xem1)

        @pl.when(sid < NST)
        def _init():
            pltpu.sync_copy(zeros_hbm.at[pl.ds(sid * RPT, RPT)],
                            acc_sh.at[pl.ds(sid * RPT, RPT)])

        iota16 = lax.iota(_i32, 16)
        ones16 = jnp.ones((16,), _f32)
        zeros16 = jnp.zeros((16,), _f32)
        onehots = [jnp.where(iota16 == jj, ones16, zeros16) for jj in range(16)]

        @pl.when(cid == 1)
        def _sc1_setup():
            def zbody(r, carry):
                for kk in range(D // 16):
                    rows_v[0, r, pl.ds(kk * 16, 16)] = zeros16
                    rows_v[1, r, pl.ds(kk * 16, 16)] = zeros16
                return carry

            lax.fori_loop(0, GB, zbody, 0)

        plsc.subcore_barrier()

        @pl.when(cid == 0)
        def _num_role():
            for s in range(S):
                ext_hbm = ehs[s]

                def fire(j, par):
                    base = sid * ETS + j * GB
                    pltpu.async_copy(ext_hbm.at[pl.ds(base, GB)],
                                     rows_v.at[par], sems[par])
                    pltpu.async_copy(
                        dst_hbm.at[pl.ds(s * ES + base, GB)],
                        didx_v.at[par], xems[par])

                def drain(j, par):
                    pltpu.make_async_copy(ext_hbm.at[pl.ds(0, GB)],
                                          rows_v.at[par], sems[par]).wait()
                    pltpu.make_async_copy(dst_hbm.at[pl.ds(0, GB)],
                                          didx_v.at[par], xems[par]).wait()
                    pltpu.sync_copy(rows_v.at[par],
                                    acc_sh.at[didx_v.at[par]], add=True)

                fire(0, 0)

                def body(k, carry):
                    j0 = 2 * k
                    fire(j0 + 1, 1)
                    drain(j0, 0)
                    fire(j0 + 2, 0)
                    drain(j0 + 1, 1)
                    return carry

                lax.fori_loop(0, GKT // 2 - 1, body, 0)
                fire(GKT - 1, 1)
                drain(GKT - 2, 0)
                drain(GKT - 1, 1)

        @pl.when(cid == 1)
        def _t_role():
            for s in range(S):
                t_hbm = ths[s]

                def fire_aux(j, par):
                    base = sid * ETS + j * GB
                    pltpu.async_copy(
                        dst_hbm.at[pl.ds(s * ES + base, GB)],
                        didx_v.at[par], xems[par])
                    pltpu.async_copy(t_hbm.at[pl.ds(base, GB)],
                                     t_v.at[par], xems[par])

                def wait_aux(par):
                    pltpu.make_async_copy(dst_hbm.at[pl.ds(0, GB)],
                                          didx_v.at[par], xems[par]).wait()
                    pltpu.make_async_copy(t_hbm.at[pl.ds(0, GB)],
                                          t_v.at[par], xems[par]).wait()

                def build(par):
                    for g in range(GB // 16):
                        tvg = t_v[par, pl.ds(g * 16, 16)]
                        for jj in range(16):
                            rows_v[par, g * 16 + jj, pl.ds(0, 16)] = (
                                tvg * onehots[jj])

                def stream(par):
                    return pltpu.async_copy(rows_v.at[par],
                                            acc_sh.at[didx_v.at[par]],
                                            sems[par], add=True)

                fire_aux(0, 0)
                wait_aux(0)
                build(0)

                def body(k, carry):
                    j0 = 2 * k
                    cp_a = stream(0)
                    fire_aux(j0 + 1, 1)
                    wait_aux(1)
                    build(1)
                    cp_a.wait()
                    cp_b = stream(1)

                    @pl.when(j0 + 2 < GKT)
                    def _prep_next():
                        fire_aux(j0 + 2, 0)
                        wait_aux(0)
                        build(0)

                    cp_b.wait()
                    return carry

                lax.fori_loop(0, GKT // 2, body, 0)

        plsc.subcore_barrier()

        @pl.when(sid < NST)
        def _writeout():
            pltpu.sync_copy(acc_sh.at[pl.ds(sid * RPT, RPT)],
                            out_hbm.at[pl.ds(cid * N + sid * RPT, RPT)])

    return scatter_k(*exts, *ts, dst_flat, zeros)


def _final_body(pn_ref, pt_ref, h_ref, bmax_ref, mW2_ref,
                mb2_ref, uW1h_ref, uW1a_ref, ub1_ref, uW2_ref, ub2_ref,
                gamma_ref, beta_ref, out_ref):
    gmax = jnp.max(bmax_ref[...])
    num = pn_ref[...]
    s0 = jnp.sum(pt_ref[:, :16], axis=1, keepdims=True)
    denom = s0 + 1e-6 * jnp.exp(gmax)
    s_agg = num / denom
    wn = s0 / denom
    agg = jnp.dot(s_agg, mW2_ref[...], preferred_element_type=_f32)
    agg = agg + wn * mb2_ref[...]
    h = h_ref[...]
    u1 = jnp.dot(h, uW1h_ref[...], preferred_element_type=_f32)
    u1 = u1 + jnp.dot(agg, uW1a_ref[...], preferred_element_type=_f32)
    u1 = jnp.maximum(u1 + ub1_ref[...], 0.0)
    out_lin = jnp.dot(u1, uW2_ref[...], preferred_element_type=_f32)
    x = jnp.maximum(out_lin + ub2_ref[...] + h, 0.0)
    mu = jnp.mean(x, axis=-1, keepdims=True)
    xc = x - mu
    var = jnp.mean(xc * xc, axis=-1, keepdims=True)
    out_ref[...] = xc * lax.rsqrt(var + 1e-5) * gamma_ref[...] + beta_ref[...]


def _final(pn, pt, h, bmax, mW2, mb2r, uW1h, uW1a, ub1r, uW2, ub2r,
           gammar, betar):
    return pl.pallas_call(
        _final_body,
        grid=(N // BN,),
        in_specs=[
            pl.BlockSpec((BN, D), lambda i: (i, 0)),
            pl.BlockSpec((BN, D), lambda i: (i, 0)),
            pl.BlockSpec((BN, D), lambda i: (i, 0)),
            pl.BlockSpec((NEB, 1, 128), lambda i: (0, 0, 0)),
            pl.BlockSpec((D, D), lambda i: (0, 0)),
            pl.BlockSpec((1, D), lambda i: (0, 0)),
            pl.BlockSpec((D, D), lambda i: (0, 0)),
            pl.BlockSpec((D, D), lambda i: (0, 0)),
            pl.BlockSpec((1, D), lambda i: (0, 0)),
            pl.BlockSpec((D, D), lambda i: (0, 0)),
            pl.BlockSpec((1, D), lambda i: (0, 0)),
            pl.BlockSpec((1, D), lambda i: (0, 0)),
            pl.BlockSpec((1, D), lambda i: (0, 0)),
        ],
        out_specs=pl.BlockSpec((BN, D), lambda i: (i, 0)),
        out_shape=jax.ShapeDtypeStruct((N, D), _f32),
    )(pn, pt, h, bmax, mW2, mb2r, uW1h, uW1a, ub1r, uW2, ub2r,
      gammar, betar)


def kernel(h, edge_index, edge_attr, mW1, mb1, mW2, mb2, aW1, ab1, aW2, ab2,
           uW1, ub1, uW2, ub2, gamma, beta):
    src4 = edge_index[0].reshape(S, NW, GKS, GB)
    dst4 = edge_index[1].reshape(S, NW, GKS, GB)

    wcat = jnp.concatenate([mW1[:D], aW1[:D], aW1[D:2 * D]], axis=1)
    bcat = jnp.concatenate([mb1, jnp.zeros((D,), _f32)]).reshape(1, 2 * D)

    b_tab, pq_tab = _tables(h, wcat, bcat)
    ab1r = ab1.reshape(1, H2)
    aW2r = aW2.reshape(1, H2)
    ab2r = ab2.reshape(1, 1)
    exts, tfs, bmaxs = [], [], []
    for s in range(S):
        bs, pqs, pqd = _gather_sc(b_tab, pq_tab, src4[s], dst4[s])
        ea_s = lax.dynamic_slice_in_dim(edge_attr, s * ES, ES, axis=0)
        ext_s, t8_s, bmax_s = _edge_stage(
            bs, pqs, pqd, ea_s, mW1[D:], aW1[2 * D:], ab1r, aW2r, ab2r)
        exts.append(ext_s)
        tfs.append(t8_s[:, 0])
        bmaxs.append(bmax_s)
    bmax = jnp.concatenate(bmaxs, axis=0)
    zeros = jnp.zeros((N, D), _f32)
    partial = _scatter_sc(exts, tfs, edge_index[1], zeros)
    out = _final(
        partial[:N], partial[N:], h, bmax, mW2,
        mb2.reshape(1, D), uW1[:D], uW1[D:], ub1.reshape(1, D), uW2,
        ub2.reshape(1, D), gamma.reshape(1, D), beta.reshape(1, D))
    return out


# speedup vs baseline: 1.0004x; 1.0004x over previous
import functools

import jax
import jax.numpy as jnp
from jax import lax
from jax.experimental import pallas as pl
from jax.experimental.pallas import tpu as pltpu
from jax.experimental.pallas import tpu_sc as plsc

N, E, D, DE = 10000, 320000, 128, 16
H2 = D // 2
NC, NS = 2, 16
NW = NC * NS
EPW = E // NW
GB = 80
GK = EPW // GB
BN = 2000
BE = 2560
NEB = E // BE
RPT = 1000
NST = N // RPT
S = 5
ES = E // S
EPWS = EPW // S
GKS = GK // S
NEBS = NEB // S
ETS = ES // NS
GKT = ETS // GB

_f32 = jnp.float32
_i32 = jnp.int32


def _tables_body(h_ref, wcat_ref, bcat_ref, b_ref, pq_ref):
    x = jnp.dot(h_ref[...], wcat_ref[...], preferred_element_type=_f32)
    x = x + bcat_ref[...]
    b_ref[...] = x[:, :D]
    pq_ref[...] = x[:, D:]


def _tables(h, wcat, bcat):
    return pl.pallas_call(
        _tables_body,
        grid=(N // BN,),
        in_specs=[
            pl.BlockSpec((BN, D), lambda i: (i, 0)),
            pl.BlockSpec((D, 2 * D), lambda i: (0, 0)),
            pl.BlockSpec((1, 2 * D), lambda i: (0, 0)),
        ],
        out_specs=[
            pl.BlockSpec((BN, D), lambda i: (i, 0)),
            pl.BlockSpec((BN, D), lambda i: (i, 0)),
        ],
        out_shape=[
            jax.ShapeDtypeStruct((N, D), _f32),
            jax.ShapeDtypeStruct((N, D), _f32),
        ],
    )(h, wcat, bcat)


def _gather_sc(b_tab, pq_tab, src3, dst3):
    mesh = plsc.VectorSubcoreMesh(core_axis_name="c", subcore_axis_name="s")

    @functools.partial(
        pl.kernel,
        mesh=mesh,
        out_type=[
            jax.ShapeDtypeStruct((ES, D), _f32),
            jax.ShapeDtypeStruct((ES, D), _f32),
            jax.ShapeDtypeStruct((ES, D), _f32),
        ],
        scratch_types=[
            pltpu.VMEM((GKS, GB), _i32),
            pltpu.VMEM((GKS, GB), _i32),
            pltpu.VMEM((2, GB, D), _f32),
            pltpu.VMEM((2, GB, D), _f32),
            pltpu.VMEM((2, GB, D), _f32),
            pltpu.SemaphoreType.DMA,
            pltpu.SemaphoreType.DMA,
        ],
    )
    def gather_k(b_hbm, pq_hbm, src_hbm, dst_hbm, bs_out, pqs_out, pqd_out,
                 src_v, dst_v, bs_v, pqs_v, pqd_v, sem0, sem1):
        wid = lax.axis_index("s") * NC + lax.axis_index("c")
        pltpu.sync_copy(src_hbm.at[wid], src_v)
        pltpu.sync_copy(dst_hbm.at[wid], dst_v)
        sems = (sem0, sem1)

        def fire(j, par):
            sem = sems[par]
            pltpu.async_copy(b_hbm.at[src_v.at[j]], bs_v.at[par], sem)
            pltpu.async_copy(pq_hbm.at[src_v.at[j]], pqs_v.at[par], sem)
            pltpu.async_copy(pq_hbm.at[dst_v.at[j]], pqd_v.at[par], sem)

        def drain(j, par):
            dummy = b_hbm.at[pl.ds(0, GB)]
            for dst in (bs_v.at[par], pqs_v.at[par], pqd_v.at[par]):
                pltpu.make_async_copy(dummy, dst, sems[par]).wait()
            base = wid * EPWS + j * GB
            pltpu.sync_copy(bs_v.at[par], bs_out.at[pl.ds(base, GB)])
            pltpu.sync_copy(pqs_v.at[par], pqs_out.at[pl.ds(base, GB)])
            pltpu.sync_copy(pqd_v.at[par], pqd_out.at[pl.ds(base, GB)])

        fire(0, 0)

        def body(k, carry):
            j0 = 2 * k
            fire(j0 + 1, 1)
            drain(j0, 0)
            fire(j0 + 2, 0)
            drain(j0 + 1, 1)
            return carry

        lax.fori_loop(0, (GKS - 1) // 2, body, 0)
        drain(GKS - 1, 0)

    return gather_k(b_tab, pq_tab, src3, dst3)


def _edge_body(bs_ref, pqs_ref, pqd_ref, ea_ref, mW1b_ref, aW1e_ref, ab1_ref,
               aW2_ref, ab2_ref, ext_ref, t8_ref, bmax_ref):
    ea = ea_ref[...]
    cem = jnp.dot(ea, mW1b_ref[...], preferred_element_type=_f32)
    cea = jnp.dot(ea, aW1e_ref[...], preferred_element_type=_f32)
    mh = jnp.maximum(bs_ref[...] + cem, 0.0)
    pre = pqs_ref[:, :H2] + pqd_ref[:, H2:] + cea + ab1_ref[...]
    ah = jnp.maximum(pre, 0.2 * pre)
    sc = jnp.sum(ah * aW2_ref[...], axis=1, keepdims=True) + ab2_ref[...]
    t = jnp.exp(sc)
    ext_ref[...] = mh * t
    t8_ref[...] = jnp.broadcast_to(t, (BE, 8))
    bmax_ref[...] = jnp.full((1, 1, 128), jnp.max(sc), dtype=_f32)


def _edge_stage(bs, pqs, pqd, ea, mW1b, aW1e, ab1r, aW2r, ab2r):
    return pl.pallas_call(
        _edge_body,
        grid=(NEBS,),
        in_specs=[
            pl.BlockSpec((BE, D), lambda i: (i, 0)),
            pl.BlockSpec((BE, D), lambda i: (i, 0)),
            pl.BlockSpec((BE, D), lambda i: (i, 0)),
            pl.BlockSpec((BE, DE), lambda i: (i, 0)),
            pl.BlockSpec((DE, D), lambda i: (0, 0)),
            pl.BlockSpec((DE, H2), lambda i: (0, 0)),
            pl.BlockSpec((1, H2), lambda i: (0, 0)),
            pl.BlockSpec((1, H2), lambda i: (0, 0)),
            pl.BlockSpec((1, 1), lambda i: (0, 0)),
        ],
        out_specs=[
            pl.BlockSpec((BE, D), lambda i: (i, 0)),
            pl.BlockSpec((BE, 8), lambda i: (i, 0)),
            pl.BlockSpec((1, 1, 128), lambda i: (i, 0, 0)),
        ],
        out_shape=[
            jax.ShapeDtypeStruct((ES, D), _f32),
            jax.ShapeDtypeStruct((ES, 8), _f32),
            jax.ShapeDtypeStruct((NEBS, 1, 128), _f32),
        ],
    )(bs, pqs, pqd, ea, mW1b, aW1e, ab1r, aW2r, ab2r)


EPT = E // NS


def _scatter_sc(exts, ts, dst_flat, zeros):
    mesh = plsc.VectorSubcoreMesh(core_axis_name="c", subcore_axis_name="s")

    @functools.partial(
        pl.kernel,
        mesh=mesh,
        out_type=jax.ShapeDtypeStruct((2 * N, D), _f32),
        scratch_types=[
            pltpu.VMEM((2, GB), _i32),
            pltpu.VMEM((2, GB, D), _f32),
            pltpu.VMEM((2, GB), _f32),
            pltpu.VMEM_SHARED((N, D), _f32),
            pltpu.SemaphoreType.DMA,
            pltpu.SemaphoreType.DMA,
            pltpu.SemaphoreType.DMA,
            pltpu.SemaphoreType.DMA,
        ],
    )
    def scatter_k(e0, e1, e2, e3, e4, t0, t1, t2, t3, t4, dst_hbm, zeros_hbm,
                  out_hbm, didx_v, rows_v, t_v, acc_sh, sem0, sem1, xem0, xem1):
        ehs = (e0, e1, e2, e3, e4)
        ths = (t0, t1, t2, t3, t4)
        cid = lax.axis_index("c")
        sid = lax.axis_index("s")
        sems = (sem0, sem1)
        xems = (xem0, xem1)

        @pl.when(sid < NST)
        def _init():
            pltpu.sync_copy(zeros_hbm.at[pl.ds(sid * RPT, RPT)],
                            acc_sh.at[pl.ds(sid * RPT, RPT)])

        iota16 = lax.iota(_i32, 16)
        ones16 = jnp.ones((16,), _f32)
        zeros16 = jnp.zeros((16,), _f32)
        onehots = [jnp.where(iota16 == jj, ones16, zeros16) for jj in range(16)]

        @pl.when(cid == 1)
        def _sc1_setup():
            def zbody(r, carry):
                for kk in range(D // 16):
                    rows_v[0, r, pl.ds(kk * 16, 16)] = zeros16
                    rows_v[1, r, pl.ds(kk * 16, 16)] = zeros16
                return carry

            lax.fori_loop(0, GB, zbody, 0)

        plsc.subcore_barrier()

        @pl.when(cid == 0)
        def _num_role():
            for s in range(S):
                ext_hbm = ehs[s]

                def fire(j, par):
                    base = sid * ETS + j * GB
                    pltpu.async_copy(ext_hbm.at[pl.ds(base, GB)],
                                     rows_v.at[par], sems[par])
                    pltpu.async_copy(
                        dst_hbm.at[pl.ds(s * ES + base, GB)],
                        didx_v.at[par], xems[par])

                def drain(j, par):
                    pltpu.make_async_copy(ext_hbm.at[pl.ds(0, GB)],
                                          rows_v.at[par], sems[par]).wait()
                    pltpu.make_async_copy(dst_hbm.at[pl.ds(0, GB)],
                                          didx_v.at[par], xems[par]).wait()
                    pltpu.sync_copy(rows_v.at[par],
                                    acc_sh.at[didx_v.at[par]], add=True)

                fire(0, 0)

                def body(k, carry):
                    j0 = 2 * k
                    fire(j0 + 1, 1)
                    drain(j0, 0)
                    fire(j0 + 2, 0)
                    drain(j0 + 1, 1)
                    return carry

                lax.fori_loop(0, GKT // 2 - 1, body, 0)
                fire(GKT - 1, 1)
                drain(GKT - 2, 0)
                drain(GKT - 1, 1)

        @pl.when(cid == 1)
        def _t_role():
            for s in range(S):
                t_hbm = ths[s]

                def fire_aux(j, par):
                    base = sid * ETS + j * GB
                    pltpu.async_copy(
                        dst_hbm.at[pl.ds(s * ES + base, GB)],
                        didx_v.at[par], xems[par])
                    pltpu.async_copy(t_hbm.at[pl.ds(base, GB)],
                                     t_v.at[par], xems[par])

                def wait_aux(par):
                    pltpu.make_async_copy(dst_hbm.at[pl.ds(0, GB)],
                                          didx_v.at[par], xems[par]).wait()
                    pltpu.make_async_copy(t_hbm.at[pl.ds(0, GB)],
                                          t_v.at[par], xems[par]).wait()

                def build(par):
                    for g in range(GB // 16):
                        tvg = t_v[par, pl.ds(g * 16, 16)]
                        for jj in range(16):
                            rows_v[par, g * 16 + jj, pl.ds(0, 16)] = (
                                tvg * onehots[jj])

                def stream(par):
                    return pltpu.async_copy(rows_v.at[par],
                                            acc_sh.at[didx_v.at[par]],
                                            sems[par], add=True)

                fire_aux(0, 0)
                wait_aux(0)
                build(0)

                def body(k, carry):
                    j0 = 2 * k
                    cp_a = stream(0)
                    fire_aux(j0 + 1, 1)
                    wait_aux(1)
                    build(1)
                    cp_a.wait()
                    cp_b = stream(1)

                    @pl.when(j0 + 2 < GKT)
                    def _prep_next():
                        fire_aux(j0 + 2, 0)
                        wait_aux(0)
                        build(0)

                    cp_b.wait()
                    return carry

                lax.fori_loop(0, GKT // 2, body, 0)

        plsc.subcore_barrier()

        @pl.when(sid < NST)
        def _writeout():
            pltpu.sync_copy(acc_sh.at[pl.ds(sid * RPT, RPT)],
                            out_hbm.at[pl.ds(cid * N + sid * RPT, RPT)])

    return scatter_k(*exts, *ts, dst_flat, zeros)


def _final_body(pn_ref, pt_ref, h_ref, bmax_ref, mW2_ref,
                mb2_ref, uW1h_ref, uW1a_ref, ub1_ref, uW2_ref, ub2_ref,
                gamma_ref, beta_ref, out_ref):
    gmax = jnp.max(bmax_ref[...])
    num = pn_ref[...]
    s0 = jnp.sum(pt_ref[:, :16], axis=1, keepdims=True)
    denom = s0 + 1e-6 * jnp.exp(gmax)
    s_agg = num / denom
    wn = s0 / denom
    agg = jnp.dot(s_agg, mW2_ref[...], preferred_element_type=_f32)
    agg = agg + wn * mb2_ref[...]
    h = h_ref[...]
    u1 = jnp.dot(h, uW1h_ref[...], preferred_element_type=_f32)
    u1 = u1 + jnp.dot(agg, uW1a_ref[...], preferred_element_type=_f32)
    u1 = jnp.maximum(u1 + ub1_ref[...], 0.0)
    out_lin = jnp.dot(u1, uW2_ref[...], preferred_element_type=_f32)
    x = jnp.maximum(out_lin + ub2_ref[...] + h, 0.0)
    mu = jnp.mean(x, axis=-1, keepdims=True)
    xc = x - mu
    var = jnp.mean(xc * xc, axis=-1, keepdims=True)
    out_ref[...] = xc * lax.rsqrt(var + 1e-5) * gamma_ref[...] + beta_ref[...]


def _final(pn, pt, h, bmax, mW2, mb2r, uW1h, uW1a, ub1r, uW2, ub2r,
           gammar, betar):
    return pl.pallas_call(
        _final_body,
        grid=(N // BN,),
        in_specs=[
            pl.BlockSpec((BN, D), lambda i: (i, 0)),
            pl.BlockSpec((BN, D), lambda i: (i, 0)),
            pl.BlockSpec((BN, D), lambda i: (i, 0)),
            pl.BlockSpec((NEB, 1, 128), lambda i: (0, 0, 0)),
            pl.BlockSpec((D, D), lambda i: (0, 0)),
            pl.BlockSpec((1, D), lambda i: (0, 0)),
            pl.BlockSpec((D, D), lambda i: (0, 0)),
            pl.BlockSpec((D, D), lambda i: (0, 0)),
            pl.BlockSpec((1, D), lambda i: (0, 0)),
            pl.BlockSpec((D, D), lambda i: (0, 0)),
            pl.BlockSpec((1, D), lambda i: (0, 0)),
            pl.BlockSpec((1, D), lambda i: (0, 0)),
            pl.BlockSpec((1, D), lambda i: (0, 0)),
        ],
        out_specs=pl.BlockSpec((BN, D), lambda i: (i, 0)),
        out_shape=jax.ShapeDtypeStruct((N, D), _f32),
    )(pn, pt, h, bmax, mW2, mb2r, uW1h, uW1a, ub1r, uW2, ub2r,
      gammar, betar)


def kernel(h, edge_index, edge_attr, mW1, mb1, mW2, mb2, aW1, ab1, aW2, ab2,
           uW1, ub1, uW2, ub2, gamma, beta):
    src4 = edge_index[0].reshape(S, NW, GKS, GB)
    dst4 = edge_index[1].reshape(S, NW, GKS, GB)

    wcat = jnp.concatenate([mW1[:D], aW1[:D], aW1[D:2 * D]], axis=1)
    bcat = jnp.concatenate([mb1, jnp.zeros((D,), _f32)]).reshape(1, 2 * D)

    b_tab, pq_tab = _tables(h, wcat, bcat)
    ab1r = ab1.reshape(1, H2)
    aW2r = aW2.reshape(1, H2)
    ab2r = ab2.reshape(1, 1)
    gath = [_gather_sc(b_tab, pq_tab, src4[s], dst4[s]) for s in range(S)]
    exts, tfs, bmaxs = [], [], []
    for s in range(S):
        bs, pqs, pqd = gath[s]
        ea_s = lax.dynamic_slice_in_dim(edge_attr, s * ES, ES, axis=0)
        ext_s, t8_s, bmax_s = _edge_stage(
            bs, pqs, pqd, ea_s, mW1[D:], aW1[2 * D:], ab1r, aW2r, ab2r)
        exts.append(ext_s)
        tfs.append(t8_s[:, 0])
        bmaxs.append(bmax_s)
    bmax = jnp.concatenate(bmaxs, axis=0)
    zeros = jnp.zeros((N, D), _f32)
    partial = _scatter_sc(exts, tfs, edge_index[1], zeros)
    out = _final(
        partial[:N], partial[N:], h, bmax, mW2,
        mb2.reshape(1, D), uW1[:D], uW1[D:], ub1.reshape(1, D), uW2,
        ub2.reshape(1, D), gamma.reshape(1, D), beta.reshape(1, D))
    return out


# speedup vs baseline: 1.0483x; 1.0479x over previous
import functools

import jax
import jax.numpy as jnp
from jax import lax
from jax.experimental import pallas as pl
from jax.experimental.pallas import tpu as pltpu
from jax.experimental.pallas import tpu_sc as plsc

N, E, D, DE = 10000, 320000, 128, 16
H2 = D // 2
NC, NS = 2, 16
NW = NC * NS
EPW = E // NW
GB = 80
GK = EPW // GB
BN = 2000
BE = 4000
NEB = E // BE
RPT = 1000
NST = N // RPT
TR = 80

_f32 = jnp.float32
_i32 = jnp.int32


def _tables_body(h_ref, wcat_ref, bcat_ref, b_ref, pq_ref):
    x = jnp.dot(h_ref[...], wcat_ref[...], preferred_element_type=_f32)
    x = x + bcat_ref[...]
    b_ref[...] = x[:, :D]
    pq_ref[...] = x[:, D:]


def _tables(h, wcat, bcat):
    return pl.pallas_call(
        _tables_body,
        grid=(N // BN,),
        in_specs=[
            pl.BlockSpec((BN, D), lambda i: (i, 0)),
            pl.BlockSpec((D, 2 * D), lambda i: (0, 0)),
            pl.BlockSpec((1, 2 * D), lambda i: (0, 0)),
        ],
        out_specs=[
            pl.BlockSpec((BN, D), lambda i: (i, 0)),
            pl.BlockSpec((BN, D), lambda i: (i, 0)),
        ],
        out_shape=[
            jax.ShapeDtypeStruct((N, D), _f32),
            jax.ShapeDtypeStruct((N, D), _f32),
        ],
    )(h, wcat, bcat)


def _gather_sc(b_tab, pq_tab, src3, dst3):
    mesh = plsc.VectorSubcoreMesh(core_axis_name="c", subcore_axis_name="s")

    @functools.partial(
        pl.kernel,
        mesh=mesh,
        out_type=[
            jax.ShapeDtypeStruct((E, D), _f32),
            jax.ShapeDtypeStruct((E, D), _f32),
            jax.ShapeDtypeStruct((E, D), _f32),
        ],
        scratch_types=[
            pltpu.VMEM((GK, GB), _i32),
            pltpu.VMEM((GK, GB), _i32),
            pltpu.VMEM((2, GB, D), _f32),
            pltpu.VMEM((2, GB, D), _f32),
            pltpu.VMEM((2, GB, D), _f32),
            pltpu.SemaphoreType.DMA,
            pltpu.SemaphoreType.DMA,
        ],
    )
    def gather_k(b_hbm, pq_hbm, src_hbm, dst_hbm, bs_out, pqs_out, pqd_out,
                 src_v, dst_v, bs_v, pqs_v, pqd_v, sem0, sem1):
        wid = lax.axis_index("s") * NC + lax.axis_index("c")
        pltpu.sync_copy(src_hbm.at[wid], src_v)
        pltpu.sync_copy(dst_hbm.at[wid], dst_v)
        sems = (sem0, sem1)

        def fire(j, par):
            sem = sems[par]
            pltpu.async_copy(b_hbm.at[src_v.at[j]], bs_v.at[par], sem)
            pltpu.async_copy(pq_hbm.at[src_v.at[j]], pqs_v.at[par], sem)
            pltpu.async_copy(pq_hbm.at[dst_v.at[j]], pqd_v.at[par], sem)

        def drain(j, par):
            dummy = b_hbm.at[pl.ds(0, GB)]
            for dst in (bs_v.at[par], pqs_v.at[par], pqd_v.at[par]):
                pltpu.make_async_copy(dummy, dst, sems[par]).wait()
            base = wid * EPW + j * GB
            pltpu.sync_copy(bs_v.at[par], bs_out.at[pl.ds(base, GB)])
            pltpu.sync_copy(pqs_v.at[par], pqs_out.at[pl.ds(base, GB)])
            pltpu.sync_copy(pqd_v.at[par], pqd_out.at[pl.ds(base, GB)])

        fire(0, 0)

        def body(k, carry):
            j0 = 2 * k
            fire(j0 + 1, 1)
            drain(j0, 0)
            fire(j0 + 2, 0)
            drain(j0 + 1, 1)
            return carry

        lax.fori_loop(0, (GK - 1) // 2, body, 0)
        drain(GK - 1, 0)

    return gather_k(b_tab, pq_tab, src3, dst3)


def _edge_body(bs_ref, pqs_ref, pqd_ref, ea_ref, mW1b_ref, aW1e_ref, ab1_ref,
               aW2_ref, ab2_ref, ext_ref, t8_ref, bmax_ref):
    ea = ea_ref[...]
    cem = jnp.dot(ea, mW1b_ref[...], preferred_element_type=_f32)
    cea = jnp.dot(ea, aW1e_ref[...], preferred_element_type=_f32)
    mh = jnp.maximum(bs_ref[...] + cem, 0.0)
    pre = pqs_ref[:, :H2] + pqd_ref[:, H2:] + cea + ab1_ref[...]
    ah = jnp.maximum(pre, 0.2 * pre)
    sc = jnp.sum(ah * aW2_ref[...], axis=1, keepdims=True) + ab2_ref[...]
    t = jnp.exp(sc)
    ext_ref[...] = mh * t
    t8_ref[...] = jnp.broadcast_to(t, (BE, 8))
    bmax_ref[...] = jnp.full((1, 1, 128), jnp.max(sc), dtype=_f32)


def _edge_stage(bs, pqs, pqd, ea, mW1b, aW1e, ab1r, aW2r, ab2r):
    return pl.pallas_call(
        _edge_body,
        grid=(NEB,),
        in_specs=[
            pl.BlockSpec((BE, D), lambda i: (i, 0)),
            pl.BlockSpec((BE, D), lambda i: (i, 0)),
            pl.BlockSpec((BE, D), lambda i: (i, 0)),
            pl.BlockSpec((BE, DE), lambda i: (i, 0)),
            pl.BlockSpec((DE, D), lambda i: (0, 0)),
            pl.BlockSpec((DE, H2), lambda i: (0, 0)),
            pl.BlockSpec((1, H2), lambda i: (0, 0)),
            pl.BlockSpec((1, H2), lambda i: (0, 0)),
            pl.BlockSpec((1, 1), lambda i: (0, 0)),
        ],
        out_specs=[
            pl.BlockSpec((BE, D), lambda i: (i, 0)),
            pl.BlockSpec((BE, 8), lambda i: (i, 0)),
            pl.BlockSpec((1, 1, 128), lambda i: (i, 0, 0)),
        ],
        out_shape=[
            jax.ShapeDtypeStruct((E, D), _f32),
            jax.ShapeDtypeStruct((E, 8), _f32),
            jax.ShapeDtypeStruct((NEB, 1, 128), _f32),
        ],
    )(bs, pqs, pqd, ea, mW1b, aW1e, ab1r, aW2r, ab2r)


EPT = E // NS
GK2 = EPT // GB


def _scatter_sc(ext, t_flat, dst3, zeros):
    mesh = plsc.VectorSubcoreMesh(core_axis_name="c", subcore_axis_name="s")

    @functools.partial(
        pl.kernel,
        mesh=mesh,
        out_type=jax.ShapeDtypeStruct((2 * N, D), _f32),
        scratch_types=[
            pltpu.VMEM((2, GB), _i32),
            pltpu.VMEM((2, GB, D), _f32),
            pltpu.VMEM((2, GB), _f32),
            pltpu.VMEM_SHARED((N, D), _f32),
            pltpu.SemaphoreType.DMA,
            pltpu.SemaphoreType.DMA,
            pltpu.SemaphoreType.DMA,
            pltpu.SemaphoreType.DMA,
        ],
    )
    def scatter_k(ext_hbm, t_hbm, dst_hbm, zeros_hbm, out_hbm,
                  didx_v, rows_v, t_v, acc_sh, sem0, sem1, xem0, xem1):
        cid = lax.axis_index("c")
        sid = lax.axis_index("s")
        sems = (sem0, sem1)
        xems = (xem0, xem1)

        @pl.when(sid < NST)
        def _init():
            pltpu.sync_copy(zeros_hbm.at[pl.ds(sid * RPT, RPT)],
                            acc_sh.at[pl.ds(sid * RPT, RPT)])

        iota16 = lax.iota(_i32, 16)
        ones16 = jnp.ones((16,), _f32)
        zeros16 = jnp.zeros((16,), _f32)
        onehots = [jnp.where(iota16 == jj, ones16, zeros16) for jj in range(16)]

        def fire_aux(j, par):
            pltpu.async_copy(dst_hbm.at[pl.ds(sid * EPT + j * GB, GB)],
                             didx_v.at[par], xems[par])
            pltpu.async_copy(t_hbm.at[pl.ds(sid * EPT + j * GB, GB)],
                             t_v.at[par], xems[par])

        def wait_aux(par):
            pltpu.make_async_copy(t_hbm.at[pl.ds(0, GB)], didx_v.at[par],
                                  xems[par]).wait()
            pltpu.make_async_copy(t_hbm.at[pl.ds(0, GB)], t_v.at[par],
                                  xems[par]).wait()

        @pl.when(cid == 1)
        def _sc1_setup():
            def zbody(r, carry):
                for k in range(D // 16):
                    rows_v[0, r, pl.ds(k * 16, 16)] = zeros16
                    rows_v[1, r, pl.ds(k * 16, 16)] = zeros16
                return carry

            lax.fori_loop(0, GB, zbody, 0)

        plsc.subcore_barrier()

        @pl.when(cid == 0)
        def _num_role():
            def fire(j, par):
                base = sid * EPT + j * GB
                pltpu.async_copy(ext_hbm.at[pl.ds(base, GB)],
                                 rows_v.at[par], sems[par])
                pltpu.async_copy(dst_hbm.at[pl.ds(sid * EPT + j * GB, GB)],
                                 didx_v.at[par], xems[par])

            def drain(j, par):
                pltpu.make_async_copy(ext_hbm.at[pl.ds(0, GB)],
                                      rows_v.at[par], sems[par]).wait()
                pltpu.make_async_copy(t_hbm.at[pl.ds(0, GB)],
                                      didx_v.at[par], xems[par]).wait()
                pltpu.sync_copy(rows_v.at[par], acc_sh.at[didx_v.at[par]],
                                add=True)

            fire(0, 0)

            def body(k, carry):
                j0 = 2 * k
                fire(j0 + 1, 1)
                drain(j0, 0)
                fire(j0 + 2, 0)
                drain(j0 + 1, 1)
                return carry

            lax.fori_loop(0, GK2 // 2 - 1, body, 0)
            fire(GK2 - 1, 1)
            drain(GK2 - 2, 0)
            drain(GK2 - 1, 1)

        @pl.when(cid == 1)
        def _t_role():
            def build(par):
                for g in range(GB // 16):
                    tvg = t_v[par, pl.ds(g * 16, 16)]
                    for jj in range(16):
                        rows_v[par, g * 16 + jj, pl.ds(0, 16)] = (
                            tvg * onehots[jj])

            def stream(par):
                return pltpu.async_copy(rows_v.at[par],
                                        acc_sh.at[didx_v.at[par]],
                                        sems[par], add=True)

            fire_aux(0, 0)
            wait_aux(0)
            build(0)

            def body(k, carry):
                j0 = 2 * k
                cp_a = stream(0)
                fire_aux(j0 + 1, 1)
                wait_aux(1)
                build(1)
                cp_a.wait()
                cp_b = stream(1)

                @pl.when(j0 + 2 < GK2)
                def _prep_next():
                    fire_aux(j0 + 2, 0)
                    wait_aux(0)
                    build(0)

                cp_b.wait()
                return carry

            lax.fori_loop(0, GK2 // 2, body, 0)

        plsc.subcore_barrier()

        @pl.when(sid < NST)
        def _writeout():
            pltpu.sync_copy(acc_sh.at[pl.ds(sid * RPT, RPT)],
                            out_hbm.at[pl.ds(cid * N + sid * RPT, RPT)])

    return scatter_k(ext, t_flat, dst3, zeros)


def _final_body(pn_ref, pt_ref, h_ref, bmax_ref, mW2_ref,
                mb2_ref, uW1h_ref, uW1a_ref, ub1_ref, uW2_ref, ub2_ref,
                gamma_ref, beta_ref, out_ref):
    gmax = jnp.max(bmax_ref[...])
    num = pn_ref[...]
    s0 = jnp.sum(pt_ref[:, :16], axis=1, keepdims=True)
    denom = s0 + 1e-6 * jnp.exp(gmax)
    s_agg = num / denom
    wn = s0 / denom
    agg = jnp.dot(s_agg, mW2_ref[...], preferred_element_type=_f32)
    agg = agg + wn * mb2_ref[...]
    h = h_ref[...]
    u1 = jnp.dot(h, uW1h_ref[...], preferred_element_type=_f32)
    u1 = u1 + jnp.dot(agg, uW1a_ref[...], preferred_element_type=_f32)
    u1 = jnp.maximum(u1 + ub1_ref[...], 0.0)
    out_lin = jnp.dot(u1, uW2_ref[...], preferred_element_type=_f32)
    x = jnp.maximum(out_lin + ub2_ref[...] + h, 0.0)
    mu = jnp.mean(x, axis=-1, keepdims=True)
    xc = x - mu
    var = jnp.mean(xc * xc, axis=-1, keepdims=True)
    out_ref[...] = xc * lax.rsqrt(var + 1e-5) * gamma_ref[...] + beta_ref[...]


def _final(pn, pt, h, bmax, mW2, mb2r, uW1h, uW1a, ub1r, uW2, ub2r,
           gammar, betar):
    return pl.pallas_call(
        _final_body,
        grid=(N // BN,),
        in_specs=[
            pl.BlockSpec((BN, D), lambda i: (i, 0)),
            pl.BlockSpec((BN, D), lambda i: (i, 0)),
            pl.BlockSpec((BN, D), lambda i: (i, 0)),
            pl.BlockSpec((NEB, 1, 128), lambda i: (0, 0, 0)),
            pl.BlockSpec((D, D), lambda i: (0, 0)),
            pl.BlockSpec((1, D), lambda i: (0, 0)),
            pl.BlockSpec((D, D), lambda i: (0, 0)),
            pl.BlockSpec((D, D), lambda i: (0, 0)),
            pl.BlockSpec((1, D), lambda i: (0, 0)),
            pl.BlockSpec((D, D), lambda i: (0, 0)),
            pl.BlockSpec((1, D), lambda i: (0, 0)),
            pl.BlockSpec((1, D), lambda i: (0, 0)),
            pl.BlockSpec((1, D), lambda i: (0, 0)),
        ],
        out_specs=pl.BlockSpec((BN, D), lambda i: (i, 0)),
        out_shape=jax.ShapeDtypeStruct((N, D), _f32),
    )(pn, pt, h, bmax, mW2, mb2r, uW1h, uW1a, ub1r, uW2, ub2r,
      gammar, betar)


def kernel(h, edge_index, edge_attr, mW1, mb1, mW2, mb2, aW1, ab1, aW2, ab2,
           uW1, ub1, uW2, ub2, gamma, beta):
    src3 = edge_index[0].reshape(NW, GK, GB)
    dst3 = edge_index[1].reshape(NW, GK, GB)

    wcat = jnp.concatenate([mW1[:D], aW1[:D], aW1[D:2 * D]], axis=1)
    bcat = jnp.concatenate([mb1, jnp.zeros((D,), _f32)]).reshape(1, 2 * D)

    b_tab, pq_tab = _tables(h, wcat, bcat)
    bs, pqs, pqd = _gather_sc(b_tab, pq_tab, src3, dst3)
    ext, t8, bmax = _edge_stage(
        bs, pqs, pqd, edge_attr, mW1[D:], aW1[2 * D:],
        ab1.reshape(1, H2), aW2.reshape(1, H2), ab2.reshape(1, 1))
    t_flat = t8[:, 0]
    zeros = jnp.zeros((N, D), _f32)
    partial = _scatter_sc(ext, t_flat, edge_index[1], zeros)
    out = _final(
        partial[:N], partial[N:], h, bmax, mW2,
        mb2.reshape(1, D), uW1[:D], uW1[D:], ub1.reshape(1, D), uW2,
        ub2.reshape(1, D), gamma.reshape(1, D), beta.reshape(1, D))
    return out


# speedup vs baseline: 1.2371x; 1.1800x over previous
import functools

import jax
import jax.numpy as jnp
from jax import lax
from jax.experimental import pallas as pl
from jax.experimental.pallas import tpu as pltpu
from jax.experimental.pallas import tpu_sc as plsc

N, E, D, DE = 10000, 320000, 128, 16
H2 = D // 2
NC, NS = 2, 16
NW = NC * NS
EPW = E // NW
GB = 80
GK = EPW // GB
BN = 2000
BE = 4000
NEB = E // BE
RPT = 1000
NST = N // RPT
TR = 80

_f32 = jnp.float32
_i32 = jnp.int32


def _tables_body(h_ref, wcat_ref, bcat_ref, tab_ref):
    x = jnp.dot(h_ref[...], wcat_ref[...], preferred_element_type=_f32)
    x = x + bcat_ref[...]
    u0 = lax.bitcast_convert_type(x[:, :D], _i32)
    u1 = lax.bitcast_convert_type(x[:, D:], _i32)
    r0 = lax.shift_right_logical(
        u0 + 0x7FFF + (lax.shift_right_logical(u0, 16) & 1), 16)
    r1 = u1 + 0x7FFF + (lax.shift_right_logical(u1, 16) & 1)
    tab_ref[...] = (r0 & 0xFFFF) | (r1 & -65536)


def _tables(h, wcat, bcat):
    return pl.pallas_call(
        _tables_body,
        grid=(N // BN,),
        in_specs=[
            pl.BlockSpec((BN, D), lambda i: (i, 0)),
            pl.BlockSpec((D, 2 * D), lambda i: (0, 0)),
            pl.BlockSpec((1, 2 * D), lambda i: (0, 0)),
        ],
        out_specs=pl.BlockSpec((BN, D), lambda i: (i, 0)),
        out_shape=jax.ShapeDtypeStruct((N, D), _i32),
    )(h, wcat, bcat)


def _gather_sc(tab, src3, dst3):
    mesh = plsc.VectorSubcoreMesh(core_axis_name="c", subcore_axis_name="s")

    @functools.partial(
        pl.kernel,
        mesh=mesh,
        out_type=[
            jax.ShapeDtypeStruct((E, D), _i32),
            jax.ShapeDtypeStruct((E, D), _i32),
        ],
        scratch_types=[
            pltpu.VMEM((GK, GB), _i32),
            pltpu.VMEM((GK, GB), _i32),
            pltpu.VMEM((2, GB, D), _i32),
            pltpu.VMEM((2, GB, D), _i32),
            pltpu.SemaphoreType.DMA,
            pltpu.SemaphoreType.DMA,
        ],
    )
    def gather_k(tab_hbm, src_hbm, dst_hbm, gs_out, gd_out,
                 src_v, dst_v, gs_v, gd_v, sem0, sem1):
        wid = lax.axis_index("s") * NC + lax.axis_index("c")
        pltpu.sync_copy(src_hbm.at[wid], src_v)
        pltpu.sync_copy(dst_hbm.at[wid], dst_v)
        sems = (sem0, sem1)

        def fire(j, par):
            sem = sems[par]
            pltpu.async_copy(tab_hbm.at[src_v.at[j]], gs_v.at[par], sem)
            pltpu.async_copy(tab_hbm.at[dst_v.at[j]], gd_v.at[par], sem)

        def drain(j, par):
            dummy = tab_hbm.at[pl.ds(0, GB)]
            for dst in (gs_v.at[par], gd_v.at[par]):
                pltpu.make_async_copy(dummy, dst, sems[par]).wait()
            base = wid * EPW + j * GB
            pltpu.sync_copy(gs_v.at[par], gs_out.at[pl.ds(base, GB)])
            pltpu.sync_copy(gd_v.at[par], gd_out.at[pl.ds(base, GB)])

        fire(0, 0)

        def body(k, carry):
            j0 = 2 * k
            fire(j0 + 1, 1)
            drain(j0, 0)
            fire(j0 + 2, 0)
            drain(j0 + 1, 1)
            return carry

        lax.fori_loop(0, (GK - 1) // 2, body, 0)
        drain(GK - 1, 0)

    return gather_k(tab, src3, dst3)


def _edge_body(gs_ref, gd_ref, ea_ref, mW1b_ref, aW1e_ref, ab1_ref,
               aW2_ref, ab2_ref, ext_ref, t8_ref, bmax_ref):
    ea = ea_ref[...]
    cem = jnp.dot(ea, mW1b_ref[...], preferred_element_type=_f32)
    cea = jnp.dot(ea, aW1e_ref[...], preferred_element_type=_f32)
    xs = gs_ref[...]
    xd = gd_ref[...]
    b_src = lax.bitcast_convert_type(lax.shift_left(xs, 16), _f32)
    pq_s = lax.bitcast_convert_type(xs & -65536, _f32)
    pq_d = lax.bitcast_convert_type(xd & -65536, _f32)
    mh = jnp.maximum(b_src + cem, 0.0)
    pre = pq_s[:, :H2] + pq_d[:, H2:] + cea + ab1_ref[...]
    ah = jnp.maximum(pre, 0.2 * pre)
    sc = jnp.sum(ah * aW2_ref[...], axis=1, keepdims=True) + ab2_ref[...]
    t = jnp.exp(sc)
    ext_ref[...] = mh * t
    t8_ref[...] = jnp.broadcast_to(t, (BE, 8))
    bmax_ref[...] = jnp.full((1, 1, 128), jnp.max(sc), dtype=_f32)


def _edge_stage(gs, gd, ea, mW1b, aW1e, ab1r, aW2r, ab2r):
    return pl.pallas_call(
        _edge_body,
        grid=(NEB,),
        in_specs=[
            pl.BlockSpec((BE, D), lambda i: (i, 0)),
            pl.BlockSpec((BE, D), lambda i: (i, 0)),
            pl.BlockSpec((BE, DE), lambda i: (i, 0)),
            pl.BlockSpec((DE, D), lambda i: (0, 0)),
            pl.BlockSpec((DE, H2), lambda i: (0, 0)),
            pl.BlockSpec((1, H2), lambda i: (0, 0)),
            pl.BlockSpec((1, H2), lambda i: (0, 0)),
            pl.BlockSpec((1, 1), lambda i: (0, 0)),
        ],
        out_specs=[
            pl.BlockSpec((BE, D), lambda i: (i, 0)),
            pl.BlockSpec((BE, 8), lambda i: (i, 0)),
            pl.BlockSpec((1, 1, 128), lambda i: (i, 0, 0)),
        ],
        out_shape=[
            jax.ShapeDtypeStruct((E, D), _f32),
            jax.ShapeDtypeStruct((E, 8), _f32),
            jax.ShapeDtypeStruct((NEB, 1, 128), _f32),
        ],
    )(gs, gd, ea, mW1b, aW1e, ab1r, aW2r, ab2r)


EPT = E // NS
GK2 = EPT // GB


def _scatter_sc(ext, t_flat, dst3, zeros):
    mesh = plsc.VectorSubcoreMesh(core_axis_name="c", subcore_axis_name="s")

    @functools.partial(
        pl.kernel,
        mesh=mesh,
        out_type=jax.ShapeDtypeStruct((2 * N, D), _f32),
        scratch_types=[
            pltpu.VMEM((2, GB), _i32),
            pltpu.VMEM((2, GB, D), _f32),
            pltpu.VMEM((2, GB), _f32),
            pltpu.VMEM_SHARED((N, D), _f32),
            pltpu.SemaphoreType.DMA,
            pltpu.SemaphoreType.DMA,
            pltpu.SemaphoreType.DMA,
            pltpu.SemaphoreType.DMA,
        ],
    )
    def scatter_k(ext_hbm, t_hbm, dst_hbm, zeros_hbm, out_hbm,
                  didx_v, rows_v, t_v, acc_sh, sem0, sem1, xem0, xem1):
        cid = lax.axis_index("c")
        sid = lax.axis_index("s")
        sems = (sem0, sem1)
        xems = (xem0, xem1)

        @pl.when(sid < NST)
        def _init():
            pltpu.sync_copy(zeros_hbm.at[pl.ds(sid * RPT, RPT)],
                            acc_sh.at[pl.ds(sid * RPT, RPT)])

        iota16 = lax.iota(_i32, 16)
        ones16 = jnp.ones((16,), _f32)
        zeros16 = jnp.zeros((16,), _f32)
        onehots = [jnp.where(iota16 == jj, ones16, zeros16) for jj in range(16)]

        def fire_aux(j, par):
            pltpu.async_copy(dst_hbm.at[pl.ds(sid * EPT + j * GB, GB)],
                             didx_v.at[par], xems[par])
            pltpu.async_copy(t_hbm.at[pl.ds(sid * EPT + j * GB, GB)],
                             t_v.at[par], xems[par])

        def wait_aux(par):
            pltpu.make_async_copy(t_hbm.at[pl.ds(0, GB)], didx_v.at[par],
                                  xems[par]).wait()
            pltpu.make_async_copy(t_hbm.at[pl.ds(0, GB)], t_v.at[par],
                                  xems[par]).wait()

        @pl.when(cid == 1)
        def _sc1_setup():
            def zbody(r, carry):
                for k in range(D // 16):
                    rows_v[0, r, pl.ds(k * 16, 16)] = zeros16
                    rows_v[1, r, pl.ds(k * 16, 16)] = zeros16
                return carry

            lax.fori_loop(0, GB, zbody, 0)

        plsc.subcore_barrier()

        @pl.when(cid == 0)
        def _num_role():
            def fire(j, par):
                base = sid * EPT + j * GB
                pltpu.async_copy(ext_hbm.at[pl.ds(base, GB)],
                                 rows_v.at[par], sems[par])
                pltpu.async_copy(dst_hbm.at[pl.ds(sid * EPT + j * GB, GB)],
                                 didx_v.at[par], xems[par])

            def drain(j, par):
                pltpu.make_async_copy(ext_hbm.at[pl.ds(0, GB)],
                                      rows_v.at[par], sems[par]).wait()
                pltpu.make_async_copy(t_hbm.at[pl.ds(0, GB)],
                                      didx_v.at[par], xems[par]).wait()
                pltpu.sync_copy(rows_v.at[par], acc_sh.at[didx_v.at[par]],
                                add=True)

            fire(0, 0)

            def body(k, carry):
                j0 = 2 * k
                fire(j0 + 1, 1)
                drain(j0, 0)
                fire(j0 + 2, 0)
                drain(j0 + 1, 1)
                return carry

            lax.fori_loop(0, GK2 // 2 - 1, body, 0)
            fire(GK2 - 1, 1)
            drain(GK2 - 2, 0)
            drain(GK2 - 1, 1)

        @pl.when(cid == 1)
        def _t_role():
            def build(par):
                for g in range(GB // 16):
                    tvg = t_v[par, pl.ds(g * 16, 16)]
                    for jj in range(16):
                        rows_v[par, g * 16 + jj, pl.ds(0, 16)] = (
                            tvg * onehots[jj])

            def stream(par):
                return pltpu.async_copy(rows_v.at[par],
                                        acc_sh.at[didx_v.at[par]],
                                        sems[par], add=True)

            fire_aux(0, 0)
            wait_aux(0)
            build(0)

            def body(k, carry):
                j0 = 2 * k
                cp_a = stream(0)
                fire_aux(j0 + 1, 1)
                wait_aux(1)
                build(1)
                cp_a.wait()
                cp_b = stream(1)

                @pl.when(j0 + 2 < GK2)
                def _prep_next():
                    fire_aux(j0 + 2, 0)
                    wait_aux(0)
                    build(0)

                cp_b.wait()
                return carry

            lax.fori_loop(0, GK2 // 2, body, 0)

        plsc.subcore_barrier()

        @pl.when(sid < NST)
        def _writeout():
            pltpu.sync_copy(acc_sh.at[pl.ds(sid * RPT, RPT)],
                            out_hbm.at[pl.ds(cid * N + sid * RPT, RPT)])

    return scatter_k(ext, t_flat, dst3, zeros)


def _final_body(pn_ref, pt_ref, h_ref, bmax_ref, mW2_ref,
                mb2_ref, uW1h_ref, uW1a_ref, ub1_ref, uW2_ref, ub2_ref,
                gamma_ref, beta_ref, out_ref):
    gmax = jnp.max(bmax_ref[...])
    num = pn_ref[...]
    s0 = jnp.sum(pt_ref[:, :16], axis=1, keepdims=True)
    denom = s0 + 1e-6 * jnp.exp(gmax)
    s_agg = num / denom
    wn = s0 / denom
    agg = jnp.dot(s_agg, mW2_ref[...], preferred_element_type=_f32)
    agg = agg + wn * mb2_ref[...]
    h = h_ref[...]
    u1 = jnp.dot(h, uW1h_ref[...], preferred_element_type=_f32)
    u1 = u1 + jnp.dot(agg, uW1a_ref[...], preferred_element_type=_f32)
    u1 = jnp.maximum(u1 + ub1_ref[...], 0.0)
    out_lin = jnp.dot(u1, uW2_ref[...], preferred_element_type=_f32)
    x = jnp.maximum(out_lin + ub2_ref[...] + h, 0.0)
    mu = jnp.mean(x, axis=-1, keepdims=True)
    xc = x - mu
    var = jnp.mean(xc * xc, axis=-1, keepdims=True)
    out_ref[...] = xc * lax.rsqrt(var + 1e-5) * gamma_ref[...] + beta_ref[...]


def _final(pn, pt, h, bmax, mW2, mb2r, uW1h, uW1a, ub1r, uW2, ub2r,
           gammar, betar):
    return pl.pallas_call(
        _final_body,
        grid=(N // BN,),
        in_specs=[
            pl.BlockSpec((BN, D), lambda i: (i, 0)),
            pl.BlockSpec((BN, D), lambda i: (i, 0)),
            pl.BlockSpec((BN, D), lambda i: (i, 0)),
            pl.BlockSpec((NEB, 1, 128), lambda i: (0, 0, 0)),
            pl.BlockSpec((D, D), lambda i: (0, 0)),
            pl.BlockSpec((1, D), lambda i: (0, 0)),
            pl.BlockSpec((D, D), lambda i: (0, 0)),
            pl.BlockSpec((D, D), lambda i: (0, 0)),
            pl.BlockSpec((1, D), lambda i: (0, 0)),
            pl.BlockSpec((D, D), lambda i: (0, 0)),
            pl.BlockSpec((1, D), lambda i: (0, 0)),
            pl.BlockSpec((1, D), lambda i: (0, 0)),
            pl.BlockSpec((1, D), lambda i: (0, 0)),
        ],
        out_specs=pl.BlockSpec((BN, D), lambda i: (i, 0)),
        out_shape=jax.ShapeDtypeStruct((N, D), _f32),
    )(pn, pt, h, bmax, mW2, mb2r, uW1h, uW1a, ub1r, uW2, ub2r,
      gammar, betar)


def kernel(h, edge_index, edge_attr, mW1, mb1, mW2, mb2, aW1, ab1, aW2, ab2,
           uW1, ub1, uW2, ub2, gamma, beta):
    src3 = edge_index[0].reshape(NW, GK, GB)
    dst3 = edge_index[1].reshape(NW, GK, GB)

    wcat = jnp.concatenate([mW1[:D], aW1[:D], aW1[D:2 * D]], axis=1)
    bcat = jnp.concatenate([mb1, jnp.zeros((D,), _f32)]).reshape(1, 2 * D)

    tab = _tables(h, wcat, bcat)
    gs, gd = _gather_sc(tab, src3, dst3)
    ext, t8, bmax = _edge_stage(
        gs, gd, edge_attr, mW1[D:], aW1[2 * D:],
        ab1.reshape(1, H2), aW2.reshape(1, H2), ab2.reshape(1, 1))
    t_flat = t8[:, 0]
    zeros = jnp.zeros((N, D), _f32)
    partial = _scatter_sc(ext, t_flat, edge_index[1], zeros)
    out = _final(
        partial[:N], partial[N:], h, bmax, mW2,
        mb2.reshape(1, D), uW1[:D], uW1[D:], ub1.reshape(1, D), uW2,
        ub2.reshape(1, D), gamma.reshape(1, D), beta.reshape(1, D))
    return out


# speedup vs baseline: 1.2466x; 1.0077x over previous
import functools

import jax
import jax.numpy as jnp
from jax import lax
from jax.experimental import pallas as pl
from jax.experimental.pallas import tpu as pltpu
from jax.experimental.pallas import tpu_sc as plsc

N, E, D, DE = 10000, 320000, 128, 16
H2 = D // 2
NC, NS = 2, 16
NW = NC * NS
EPW = E // NW
GB = 80
GK = EPW // GB
BN = 2000
BE = 8000
NEB = E // BE
RPT = 1000
NST = N // RPT
TR = 80

_f32 = jnp.float32
_i32 = jnp.int32


def _tables_body(h_ref, wcat_ref, bcat_ref, tab_ref):
    x = jnp.dot(h_ref[...], wcat_ref[...], preferred_element_type=_f32)
    x = x + bcat_ref[...]
    u0 = lax.bitcast_convert_type(x[:, :D], _i32)
    u1 = lax.bitcast_convert_type(x[:, D:], _i32)
    r0 = lax.shift_right_logical(
        u0 + 0x7FFF + (lax.shift_right_logical(u0, 16) & 1), 16)
    r1 = u1 + 0x7FFF + (lax.shift_right_logical(u1, 16) & 1)
    tab_ref[...] = (r0 & 0xFFFF) | (r1 & -65536)


def _tables(h, wcat, bcat):
    return pl.pallas_call(
        _tables_body,
        grid=(N // BN,),
        in_specs=[
            pl.BlockSpec((BN, D), lambda i: (i, 0)),
            pl.BlockSpec((D, 2 * D), lambda i: (0, 0)),
            pl.BlockSpec((1, 2 * D), lambda i: (0, 0)),
        ],
        out_specs=pl.BlockSpec((BN, D), lambda i: (i, 0)),
        out_shape=jax.ShapeDtypeStruct((N, D), _i32),
    )(h, wcat, bcat)


def _gather_sc(tab, src3, dst3):
    mesh = plsc.VectorSubcoreMesh(core_axis_name="c", subcore_axis_name="s")

    @functools.partial(
        pl.kernel,
        mesh=mesh,
        out_type=[
            jax.ShapeDtypeStruct((E, D), _i32),
            jax.ShapeDtypeStruct((E, D), _i32),
        ],
        scratch_types=[
            pltpu.VMEM((GK, GB), _i32),
            pltpu.VMEM((GK, GB), _i32),
            pltpu.VMEM((2, GB, D), _i32),
            pltpu.VMEM((2, GB, D), _i32),
            pltpu.SemaphoreType.DMA,
            pltpu.SemaphoreType.DMA,
        ],
    )
    def gather_k(tab_hbm, src_hbm, dst_hbm, gs_out, gd_out,
                 src_v, dst_v, gs_v, gd_v, sem0, sem1):
        wid = lax.axis_index("s") * NC + lax.axis_index("c")
        pltpu.sync_copy(src_hbm.at[wid], src_v)
        pltpu.sync_copy(dst_hbm.at[wid], dst_v)
        sems = (sem0, sem1)

        def fire(j, par):
            sem = sems[par]
            pltpu.async_copy(tab_hbm.at[src_v.at[j]], gs_v.at[par], sem)
            pltpu.async_copy(tab_hbm.at[dst_v.at[j]], gd_v.at[par], sem)

        def drain(j, par):
            dummy = tab_hbm.at[pl.ds(0, GB)]
            for dst in (gs_v.at[par], gd_v.at[par]):
                pltpu.make_async_copy(dummy, dst, sems[par]).wait()
            base = wid * EPW + j * GB
            pltpu.sync_copy(gs_v.at[par], gs_out.at[pl.ds(base, GB)])
            pltpu.sync_copy(gd_v.at[par], gd_out.at[pl.ds(base, GB)])

        fire(0, 0)

        def body(k, carry):
            j0 = 2 * k
            fire(j0 + 1, 1)
            drain(j0, 0)
            fire(j0 + 2, 0)
            drain(j0 + 1, 1)
            return carry

        lax.fori_loop(0, (GK - 1) // 2, body, 0)
        drain(GK - 1, 0)

    return gather_k(tab, src3, dst3)


def _edge_body(gs_ref, gd_ref, ea_ref, mW1b_ref, aW1e_ref, ab1_ref,
               aW2_ref, ab2_ref, ext_ref, t8_ref, bmax_ref):
    ea = ea_ref[...]
    cem = jnp.dot(ea, mW1b_ref[...], preferred_element_type=_f32)
    cea = jnp.dot(ea, aW1e_ref[...], preferred_element_type=_f32)
    xs = gs_ref[...]
    xd = gd_ref[...]
    b_src = lax.bitcast_convert_type(lax.shift_left(xs, 16), _f32)
    pq_s = lax.bitcast_convert_type(xs & -65536, _f32)
    pq_d = lax.bitcast_convert_type(xd & -65536, _f32)
    mh = jnp.maximum(b_src + cem, 0.0)
    pre = pq_s[:, :H2] + pq_d[:, H2:] + cea + ab1_ref[...]
    ah = jnp.maximum(pre, 0.2 * pre)
    sc = jnp.sum(ah * aW2_ref[...], axis=1, keepdims=True) + ab2_ref[...]
    t = jnp.exp(sc)
    ext_ref[...] = mh * t
    t8_ref[...] = jnp.broadcast_to(t, (BE, 8))
    bmax_ref[...] = jnp.full((1, 1, 128), jnp.max(sc), dtype=_f32)


def _edge_stage(gs, gd, ea, mW1b, aW1e, ab1r, aW2r, ab2r):
    return pl.pallas_call(
        _edge_body,
        grid=(NEB,),
        in_specs=[
            pl.BlockSpec((BE, D), lambda i: (i, 0)),
            pl.BlockSpec((BE, D), lambda i: (i, 0)),
            pl.BlockSpec((BE, DE), lambda i: (i, 0)),
            pl.BlockSpec((DE, D), lambda i: (0, 0)),
            pl.BlockSpec((DE, H2), lambda i: (0, 0)),
            pl.BlockSpec((1, H2), lambda i: (0, 0)),
            pl.BlockSpec((1, H2), lambda i: (0, 0)),
            pl.BlockSpec((1, 1), lambda i: (0, 0)),
        ],
        out_specs=[
            pl.BlockSpec((BE, D), lambda i: (i, 0)),
            pl.BlockSpec((BE, 8), lambda i: (i, 0)),
            pl.BlockSpec((1, 1, 128), lambda i: (i, 0, 0)),
        ],
        out_shape=[
            jax.ShapeDtypeStruct((E, D), _f32),
            jax.ShapeDtypeStruct((E, 8), _f32),
            jax.ShapeDtypeStruct((NEB, 1, 128), _f32),
        ],
    )(gs, gd, ea, mW1b, aW1e, ab1r, aW2r, ab2r)


EPT = E // NS
GK2 = EPT // GB


def _scatter_sc(ext, t_flat, dst3, zeros):
    mesh = plsc.VectorSubcoreMesh(core_axis_name="c", subcore_axis_name="s")

    @functools.partial(
        pl.kernel,
        mesh=mesh,
        out_type=jax.ShapeDtypeStruct((2 * N, D), _f32),
        scratch_types=[
            pltpu.VMEM((2, GB), _i32),
            pltpu.VMEM((2, GB, D), _f32),
            pltpu.VMEM((2, GB), _f32),
            pltpu.VMEM_SHARED((N, D), _f32),
            pltpu.SemaphoreType.DMA,
            pltpu.SemaphoreType.DMA,
            pltpu.SemaphoreType.DMA,
            pltpu.SemaphoreType.DMA,
        ],
    )
    def scatter_k(ext_hbm, t_hbm, dst_hbm, zeros_hbm, out_hbm,
                  didx_v, rows_v, t_v, acc_sh, sem0, sem1, xem0, xem1):
        cid = lax.axis_index("c")
        sid = lax.axis_index("s")
        sems = (sem0, sem1)
        xems = (xem0, xem1)

        @pl.when(sid < NST)
        def _init():
            pltpu.sync_copy(zeros_hbm.at[pl.ds(sid * RPT, RPT)],
                            acc_sh.at[pl.ds(sid * RPT, RPT)])

        iota16 = lax.iota(_i32, 16)
        ones16 = jnp.ones((16,), _f32)
        zeros16 = jnp.zeros((16,), _f32)
        onehots = [jnp.where(iota16 == jj, ones16, zeros16) for jj in range(16)]

        def fire_aux(j, par):
            pltpu.async_copy(dst_hbm.at[pl.ds(sid * EPT + j * GB, GB)],
                             didx_v.at[par], xems[par])
            pltpu.async_copy(t_hbm.at[pl.ds(sid * EPT + j * GB, GB)],
                             t_v.at[par], xems[par])

        def wait_aux(par):
            pltpu.make_async_copy(t_hbm.at[pl.ds(0, GB)], didx_v.at[par],
                                  xems[par]).wait()
            pltpu.make_async_copy(t_hbm.at[pl.ds(0, GB)], t_v.at[par],
                                  xems[par]).wait()

        @pl.when(cid == 1)
        def _sc1_setup():
            def zbody(r, carry):
                for k in range(D // 16):
                    rows_v[0, r, pl.ds(k * 16, 16)] = zeros16
                    rows_v[1, r, pl.ds(k * 16, 16)] = zeros16
                return carry

            lax.fori_loop(0, GB, zbody, 0)

        plsc.subcore_barrier()

        @pl.when(cid == 0)
        def _num_role():
            def fire(j, par):
                base = sid * EPT + j * GB
                pltpu.async_copy(ext_hbm.at[pl.ds(base, GB)],
                                 rows_v.at[par], sems[par])
                pltpu.async_copy(dst_hbm.at[pl.ds(sid * EPT + j * GB, GB)],
                                 didx_v.at[par], xems[par])

            def drain(j, par):
                pltpu.make_async_copy(ext_hbm.at[pl.ds(0, GB)],
                                      rows_v.at[par], sems[par]).wait()
                pltpu.make_async_copy(t_hbm.at[pl.ds(0, GB)],
                                      didx_v.at[par], xems[par]).wait()
                pltpu.sync_copy(rows_v.at[par], acc_sh.at[didx_v.at[par]],
                                add=True)

            fire(0, 0)

            def body(k, carry):
                j0 = 2 * k
                fire(j0 + 1, 1)
                drain(j0, 0)
                fire(j0 + 2, 0)
                drain(j0 + 1, 1)
                return carry

            lax.fori_loop(0, GK2 // 2 - 1, body, 0)
            fire(GK2 - 1, 1)
            drain(GK2 - 2, 0)
            drain(GK2 - 1, 1)

        @pl.when(cid == 1)
        def _t_role():
            def build(par):
                for g in range(GB // 16):
                    tvg = t_v[par, pl.ds(g * 16, 16)]
                    for jj in range(16):
                        rows_v[par, g * 16 + jj, pl.ds(0, 16)] = (
                            tvg * onehots[jj])

            def stream(par):
                return pltpu.async_copy(rows_v.at[par],
                                        acc_sh.at[didx_v.at[par]],
                                        sems[par], add=True)

            fire_aux(0, 0)
            wait_aux(0)
            build(0)

            def body(k, carry):
                j0 = 2 * k
                cp_a = stream(0)
                fire_aux(j0 + 1, 1)
                wait_aux(1)
                build(1)
                cp_a.wait()
                cp_b = stream(1)

                @pl.when(j0 + 2 < GK2)
                def _prep_next():
                    fire_aux(j0 + 2, 0)
                    wait_aux(0)
                    build(0)

                cp_b.wait()
                return carry

            lax.fori_loop(0, GK2 // 2, body, 0)

        plsc.subcore_barrier()

        @pl.when(sid < NST)
        def _writeout():
            pltpu.sync_copy(acc_sh.at[pl.ds(sid * RPT, RPT)],
                            out_hbm.at[pl.ds(cid * N + sid * RPT, RPT)])

    return scatter_k(ext, t_flat, dst3, zeros)


def _final_body(pn_ref, pt_ref, h_ref, bmax_ref, mW2_ref,
                mb2_ref, uW1h_ref, uW1a_ref, ub1_ref, uW2_ref, ub2_ref,
                gamma_ref, beta_ref, out_ref):
    gmax = jnp.max(bmax_ref[...])
    num = pn_ref[...]
    s0 = jnp.sum(pt_ref[:, :16], axis=1, keepdims=True)
    denom = s0 + 1e-6 * jnp.exp(gmax)
    s_agg = num / denom
    wn = s0 / denom
    agg = jnp.dot(s_agg, mW2_ref[...], preferred_element_type=_f32)
    agg = agg + wn * mb2_ref[...]
    h = h_ref[...]
    u1 = jnp.dot(h, uW1h_ref[...], preferred_element_type=_f32)
    u1 = u1 + jnp.dot(agg, uW1a_ref[...], preferred_element_type=_f32)
    u1 = jnp.maximum(u1 + ub1_ref[...], 0.0)
    out_lin = jnp.dot(u1, uW2_ref[...], preferred_element_type=_f32)
    x = jnp.maximum(out_lin + ub2_ref[...] + h, 0.0)
    mu = jnp.mean(x, axis=-1, keepdims=True)
    xc = x - mu
    var = jnp.mean(xc * xc, axis=-1, keepdims=True)
    out_ref[...] = xc * lax.rsqrt(var + 1e-5) * gamma_ref[...] + beta_ref[...]


def _final(pn, pt, h, bmax, mW2, mb2r, uW1h, uW1a, ub1r, uW2, ub2r,
           gammar, betar):
    return pl.pallas_call(
        _final_body,
        grid=(N // BN,),
        in_specs=[
            pl.BlockSpec((BN, D), lambda i: (i, 0)),
            pl.BlockSpec((BN, D), lambda i: (i, 0)),
            pl.BlockSpec((BN, D), lambda i: (i, 0)),
            pl.BlockSpec((NEB, 1, 128), lambda i: (0, 0, 0)),
            pl.BlockSpec((D, D), lambda i: (0, 0)),
            pl.BlockSpec((1, D), lambda i: (0, 0)),
            pl.BlockSpec((D, D), lambda i: (0, 0)),
            pl.BlockSpec((D, D), lambda i: (0, 0)),
            pl.BlockSpec((1, D), lambda i: (0, 0)),
            pl.BlockSpec((D, D), lambda i: (0, 0)),
            pl.BlockSpec((1, D), lambda i: (0, 0)),
            pl.BlockSpec((1, D), lambda i: (0, 0)),
            pl.BlockSpec((1, D), lambda i: (0, 0)),
        ],
        out_specs=pl.BlockSpec((BN, D), lambda i: (i, 0)),
        out_shape=jax.ShapeDtypeStruct((N, D), _f32),
    )(pn, pt, h, bmax, mW2, mb2r, uW1h, uW1a, ub1r, uW2, ub2r,
      gammar, betar)


def kernel(h, edge_index, edge_attr, mW1, mb1, mW2, mb2, aW1, ab1, aW2, ab2,
           uW1, ub1, uW2, ub2, gamma, beta):
    src3 = edge_index[0].reshape(NW, GK, GB)
    dst3 = edge_index[1].reshape(NW, GK, GB)

    wcat = jnp.concatenate([mW1[:D], aW1[:D], aW1[D:2 * D]], axis=1)
    bcat = jnp.concatenate([mb1, jnp.zeros((D,), _f32)]).reshape(1, 2 * D)

    tab = _tables(h, wcat, bcat)
    gs, gd = _gather_sc(tab, src3, dst3)
    ext, t8, bmax = _edge_stage(
        gs, gd, edge_attr, mW1[D:], aW1[2 * D:],
        ab1.reshape(1, H2), aW2.reshape(1, H2), ab2.reshape(1, 1))
    t_flat = t8[:, 0]
    zeros = jnp.zeros((N, D), _f32)
    partial = _scatter_sc(ext, t_flat, edge_index[1], zeros)
    out = _final(
        partial[:N], partial[N:], h, bmax, mW2,
        mb2.reshape(1, D), uW1[:D], uW1[D:], ub1.reshape(1, D), uW2,
        ub2.reshape(1, D), gamma.reshape(1, D), beta.reshape(1, D))
    return out
